# baseline (speedup 1.0000x reference)
import jax
import jax.numpy as jnp
from jax.experimental import pallas as pl


def _copy_body(x_ref, o_ref):
    o_ref[...] = x_ref[...]


def _pallas_copy(x, block_rows):
    rows, cols = x.shape
    return pl.pallas_call(
        _copy_body,
        grid=(rows // block_rows,),
        in_specs=[pl.BlockSpec((block_rows, cols), lambda i: (i, 0))],
        out_specs=pl.BlockSpec((block_rows, cols), lambda i: (i, 0)),
        out_shape=jax.ShapeDtypeStruct(x.shape, x.dtype),
    )(x)


def kernel(x_dict, edge_index, entity_emb, rel_emb):
    entity_out = _pallas_copy(entity_emb, 10000)
    rel_out = _pallas_copy(rel_emb, 512)
    return (entity_out, rel_out)
